# Initial kernel scaffold; baseline (speedup 1.0000x reference)
import jax
import jax.numpy as jnp
from jax.experimental import pallas as pl


def kernel(x, weights, biases, responses):
    raise NotImplementedError("write your pallas kernel here")


# speedup vs baseline: 2.0063x; 2.0063x over previous
import functools

import jax
import jax.numpy as jnp
from jax import lax
from jax.experimental import pallas as pl
from jax.experimental.pallas import tpu as pltpu
from jax.experimental.pallas import tpu_sc as plsc

N_IN = 16
N_H1 = 16
N_H2 = 8
N_OUT = 4
BATCH = 16384
L = 16
NW = 32
ROWS = BATCH // NW
GROUPS = ROWS // L

_MESH = plsc.VectorSubcoreMesh(core_axis_name="c", subcore_axis_name="s")


def _stanh(a):
    e = jnp.exp(-2.0 * jnp.abs(a))
    t = (1.0 - e) / (1.0 + e)
    return jnp.where(a < 0.0, -t, t)


@functools.partial(
    pl.kernel,
    out_type=jax.ShapeDtypeStruct((NW, ROWS * N_OUT), jnp.float32),
    mesh=_MESH,
    scratch_types=[
        pltpu.VMEM((ROWS * N_IN,), jnp.float32),
        pltpu.VMEM((ROWS * N_OUT,), jnp.float32),
        pltpu.VMEM((128,), jnp.float32),
        pltpu.VMEM((32,), jnp.float32),
        pltpu.VMEM((32,), jnp.float32),
    ],
    compiler_params=pltpu.CompilerParams(needs_layout_passes=False),
)
def _neat(x_hbm, w_hbm, b_hbm, r_hbm, out_hbm, xv, ov, wv, bv, rv):
    wid = lax.axis_index("s") * 2 + lax.axis_index("c")
    pltpu.sync_copy(x_hbm.at[wid], xv)
    pltpu.sync_copy(w_hbm, wv)
    pltpu.sync_copy(b_hbm, bv)
    pltpu.sync_copy(r_hbm, rv)

    wvecs = [wv[pl.ds(i * L, L)] for i in range(128 // L)]
    bvecs = [bv[pl.ds(i * L, L)] for i in range(2)]
    rvecs = [rv[pl.ds(i * L, L)] for i in range(2)]
    ws = [wvecs[i // L][i % L] for i in range(128)]
    bs = [bvecs[i // L][i % L] for i in range(28)]
    rs = [rvecs[i // L][i % L] for i in range(28)]

    iota = lax.iota(jnp.int32, L)

    def body(g, carry):
        xbase = g * (L * N_IN) + iota * N_IN
        cols = [plsc.load_gather(xv, [xbase + c]) for c in range(N_IN)]
        h1 = []
        for h in range(N_H1):
            a = cols[h % 16] * ws[4 * h]
            for k in range(1, 4):
                a = a + cols[(h + k) % 16] * ws[4 * h + k]
            h1.append(_stanh(a + bs[h]) * rs[h])
        h2 = []
        for j in range(N_H2):
            a = h1[(2 * j) % 16] * ws[64 + 4 * j]
            for k in range(1, 4):
                a = a + h1[(2 * j + k) % 16] * ws[64 + 4 * j + k]
            h2.append(_stanh(a + bs[16 + j]) * rs[16 + j])
        obase = g * (L * N_OUT) + iota * N_OUT
        for o in range(N_OUT):
            a = h2[0] * ws[96 + 8 * o]
            for s in range(1, 8):
                a = a + h2[s] * ws[96 + 8 * o + s]
            val = _stanh(a + bs[24 + o]) * rs[24 + o]
            plsc.store_scatter(ov, [obase + o], val)
        return carry

    lax.fori_loop(0, GROUPS, body, 0)
    pltpu.sync_copy(ov, out_hbm.at[wid])


def kernel(x, weights, biases, responses):
    b_pad = jnp.pad(biases, (0, 4))
    r_pad = jnp.pad(responses, (0, 4))
    x_flat = x.reshape(NW, ROWS * N_IN)
    out = _neat(x_flat, weights, b_pad, r_pad)
    return out.reshape(BATCH, N_OUT)


# speedup vs baseline: 2.2296x; 1.1113x over previous
import functools

import jax
import jax.numpy as jnp
from jax import lax
from jax.experimental import pallas as pl
from jax.experimental.pallas import tpu as pltpu
from jax.experimental.pallas import tpu_sc as plsc

N_IN = 16
N_H1 = 16
N_H2 = 8
N_OUT = 4
BATCH = 16384
L = 16
NW = 32
ROWS = BATCH // NW
HALF = ROWS // 2
GROUPS_H = HALF // L

NP = 164
W1_, B1_, W2_, B2_, W3_, B3_, R3A_, R3B_ = 0, 64, 80, 112, 120, 152, 156, 160

_MESH = plsc.VectorSubcoreMesh(core_axis_name="c", subcore_axis_name="s")


@functools.partial(
    pl.kernel,
    out_type=jax.ShapeDtypeStruct((BATCH, N_OUT), jnp.float32),
    mesh=_MESH,
    scratch_types=[
        pltpu.VMEM((HALF, N_IN), jnp.float32),
        pltpu.VMEM((ROWS, N_OUT), jnp.float32),
        pltpu.VMEM((NP * L,), jnp.float32),
    ],
    compiler_params=pltpu.CompilerParams(needs_layout_passes=False),
)
def _neat(x_hbm, p_hbm, out_hbm, xv, ov, pv):
    wid = lax.axis_index("s") * 2 + lax.axis_index("c")
    x4 = x_hbm.reshape(NW, 2, HALF, N_IN)
    pltpu.sync_copy(p_hbm, pv)

    iota = lax.iota(jnp.int32, L)

    def P(i):
        return pv[pl.ds(i * L, L)]

    for half in range(2):
        pltpu.sync_copy(x4.at[wid, half], xv)

        def body(g, carry):
            rows = g * L + iota
            cols = [
                plsc.load_gather(xv, [rows, jnp.full((L,), c, jnp.int32)])
                for c in range(N_IN)
            ]
            h1 = []
            for h in range(N_H1):
                a = cols[h % 16] * P(W1_ + 4 * h)
                for k in range(1, 4):
                    a = a + cols[(h + k) % 16] * P(W1_ + 4 * h + k)
                d = jnp.exp(a + P(B1_ + h)) + 1.0
                h1.append(1.0 - 2.0 / d)
            h2 = []
            for j in range(N_H2):
                a = h1[(2 * j) % 16] * P(W2_ + 4 * j)
                for k in range(1, 4):
                    a = a + h1[(2 * j + k) % 16] * P(W2_ + 4 * j + k)
                d = jnp.exp(a + P(B2_ + j)) + 1.0
                h2.append(1.0 - 2.0 / d)
            orows = half * HALF + rows
            for o in range(N_OUT):
                a = h2[0] * P(W3_ + 8 * o)
                for s in range(1, 8):
                    a = a + h2[s] * P(W3_ + 8 * o + s)
                d = jnp.exp(a + P(B3_ + o)) + 1.0
                val = P(R3A_ + o) - P(R3B_ + o) / d
                plsc.store_scatter(ov, [orows, jnp.full((L,), o, jnp.int32)], val)
            return carry

        lax.fori_loop(0, GROUPS_H, body, 0)

    pltpu.sync_copy(ov, out_hbm.reshape(NW, ROWS, N_OUT).at[wid])


def kernel(x, weights, biases, responses):
    r1 = responses[:16]
    r2 = responses[16:24]
    w1 = 2.0 * weights[:64]
    b1 = 2.0 * biases[:16]
    src2 = (2 * jnp.arange(8)[:, None] + jnp.arange(4)[None, :]) % 16
    w2 = (2.0 * weights[64:96].reshape(8, 4) * r1[src2]).reshape(32)
    b2 = 2.0 * biases[16:24]
    w3 = (2.0 * weights[96:128].reshape(4, 8) * r2[None, :]).reshape(32)
    b3 = 2.0 * biases[24:28]
    r3a = responses[24:28]
    r3b = 2.0 * responses[24:28]
    params = jnp.concatenate([w1, b1, w2, b2, w3, b3, r3a, r3b])
    ptab = jnp.repeat(params, L)
    return _neat(x, ptab)
